# baseline (speedup 1.0000x reference)
import functools

import jax
import jax.numpy as jnp
from jax import lax
from jax.experimental import pallas as pl
from jax.experimental.pallas import tpu as pltpu
from jax.experimental.pallas import tpu_sc as plsc

NC, NS, LANES = 2, 16, 16
NW = NC * NS
CH = 128


def _edge_aggregate(g, src3, dst3, zeros2d, n_pad):
    n_rows = src3.shape[1]
    d = g.shape[1]
    rows_per_tile = n_pad // NS
    half = n_rows // 2
    assert n_rows % 4 == 0

    mesh = plsc.VectorSubcoreMesh(core_axis_name="c", subcore_axis_name="s")

    @functools.partial(
        pl.kernel,
        out_type=jax.ShapeDtypeStruct((NC, n_pad, d), jnp.float32),
        mesh=mesh,
        scratch_types=[
            pltpu.VMEM((n_rows, CH), jnp.int32),
            pltpu.VMEM((half, CH), jnp.int32),
            pltpu.VMEM((CH, d), jnp.float32),
            pltpu.VMEM((CH, d), jnp.float32),
            pltpu.VMEM_SHARED((n_pad, d), jnp.float32),
            pltpu.SemaphoreType.DMA,
            pltpu.SemaphoreType.DMA,
        ],
    )
    def body(g_hbm, src_hbm, dst_hbm, z_hbm, out_hbm, srcv, dstv,
             bufa, bufb, acc, sema, semb):
        c = lax.axis_index("c")
        s = lax.axis_index("s")
        wid = c * NS + s
        pltpu.sync_copy(src_hbm.at[wid], srcv)
        pltpu.sync_copy(dst_hbm.at[wid, pl.ds(0, half)], dstv)
        base = s * rows_per_tile
        pltpu.sync_copy(z_hbm, acc.at[pl.ds(base, rows_per_tile)])
        plsc.subcore_barrier()

        def fire(j, buf, sem):
            pltpu.async_copy(g_hbm.at[srcv.at[j]], buf, sem)

        def drain(j, buf, sem):
            pltpu.make_async_copy(g_hbm.at[srcv.at[j]], buf, sem).wait()

        def half_loop(j_base):
            def step(j2, carry):
                j = j_base + 2 * j2
                fire(j + 1, bufb, semb)
                drain(j, bufa, sema)
                pltpu.sync_copy(bufa, acc.at[dstv.at[2 * j2]], add=True)

                @pl.when(j + 2 < n_rows)
                def _():
                    fire(j + 2, bufa, sema)

                drain(j + 1, bufb, semb)
                pltpu.sync_copy(bufb, acc.at[dstv.at[2 * j2 + 1]], add=True)
                return carry

            lax.fori_loop(0, half // 2, step, 0)

        fire(0, bufa, sema)
        half_loop(0)
        pltpu.sync_copy(dst_hbm.at[wid, pl.ds(half, half)], dstv)
        half_loop(half)

        plsc.subcore_barrier()
        pltpu.sync_copy(acc.at[pl.ds(base, rows_per_tile)],
                        out_hbm.at[c, pl.ds(base, rows_per_tile)])

    return body(g, src3, dst3, zeros2d)


def _degree(dst3, zeros1d, ones1d, n_pad):
    n_rows = dst3.shape[1]
    rows_per_tile = n_pad // NS

    mesh = plsc.VectorSubcoreMesh(core_axis_name="c", subcore_axis_name="s")

    @functools.partial(
        pl.kernel,
        out_type=jax.ShapeDtypeStruct((NC, n_pad), jnp.float32),
        mesh=mesh,
        scratch_types=[
            pltpu.VMEM((n_rows, CH), jnp.int32),
            pltpu.VMEM((CH,), jnp.float32),
            pltpu.VMEM_SHARED((n_pad,), jnp.float32),
        ],
    )
    def body(dst_hbm, z_hbm, ones_hbm, out_hbm, dstv, onesv, acc):
        c = lax.axis_index("c")
        s = lax.axis_index("s")
        wid = c * NS + s
        pltpu.sync_copy(ones_hbm, onesv)
        pltpu.sync_copy(dst_hbm.at[wid], dstv)
        base = s * rows_per_tile
        pltpu.sync_copy(z_hbm, acc.at[pl.ds(base, rows_per_tile)])
        plsc.subcore_barrier()

        def chunk(j, carry):
            pltpu.sync_copy(onesv, acc.at[dstv.at[j]], add=True)
            return carry

        lax.fori_loop(0, n_rows, chunk, 0)
        plsc.subcore_barrier()
        pltpu.sync_copy(acc.at[pl.ds(base, rows_per_tile)],
                        out_hbm.at[c, pl.ds(base, rows_per_tile)])

    return body(dst3, zeros1d, ones1d)


_BR = 1000


def _tc_prep(x, W1, degp):
    n, d = x.shape

    def body(x_ref, w_ref, degp_ref, g_ref, dinv_ref):
        deg = degp_ref[0] + degp_ref[1] + 1.0
        dinv = 1.0 / jnp.sqrt(deg)
        dinv_ref[...] = dinv
        g_ref[...] = jnp.dot(x_ref[...], w_ref[...],
                             preferred_element_type=jnp.float32) * dinv

    return pl.pallas_call(
        body,
        grid=(n // _BR,),
        in_specs=[
            pl.BlockSpec((_BR, d), lambda i: (i, 0)),
            pl.BlockSpec((d, d), lambda i: (0, 0)),
            pl.BlockSpec((NC, _BR, 1), lambda i: (0, i, 0)),
        ],
        out_specs=[
            pl.BlockSpec((_BR, d), lambda i: (i, 0)),
            pl.BlockSpec((_BR, 1), lambda i: (i, 0)),
        ],
        out_shape=[
            jax.ShapeDtypeStruct((n, d), jnp.float32),
            jax.ShapeDtypeStruct((n, 1), jnp.float32),
        ],
    )(x, W1, degp)


def _tc_mid(parts, g1, dinv, b1, W2):
    n, d = g1.shape

    def body(p_ref, g_ref, dinv_ref, b_ref, w_ref, out_ref):
        sm = p_ref[0] + p_ref[1] + g_ref[...]
        x2 = jnp.maximum(sm * dinv_ref[...] + b_ref[...], 0.0)
        out_ref[...] = jnp.dot(x2, w_ref[...],
                               preferred_element_type=jnp.float32) * dinv_ref[...]

    return pl.pallas_call(
        body,
        grid=(n // _BR,),
        in_specs=[
            pl.BlockSpec((NC, _BR, d), lambda i: (0, i, 0)),
            pl.BlockSpec((_BR, d), lambda i: (i, 0)),
            pl.BlockSpec((_BR, 1), lambda i: (i, 0)),
            pl.BlockSpec((1, d), lambda i: (0, 0)),
            pl.BlockSpec((d, d), lambda i: (0, 0)),
        ],
        out_specs=pl.BlockSpec((_BR, d), lambda i: (i, 0)),
        out_shape=jax.ShapeDtypeStruct((n, d), jnp.float32),
    )(parts, g1, dinv, b1, W2)


def _tc_final(parts, g2, dinv, b2, Wf, x, Ws, bf, bs):
    n, d = g2.shape
    dout = Wf.shape[1]

    def body(p_ref, g_ref, dinv_ref, b2_ref, wf_ref, x_ref, ws_ref,
             bf_ref, bs_ref, out_ref):
        sm = p_ref[0] + p_ref[1] + g_ref[...]
        h2 = jnp.maximum(sm * dinv_ref[...] + b2_ref[...], 0.0)
        out_ref[...] = (jnp.dot(h2, wf_ref[...], preferred_element_type=jnp.float32)
                        + jnp.dot(x_ref[...], ws_ref[...],
                                  preferred_element_type=jnp.float32)
                        + bf_ref[...] + bs_ref[...])

    return pl.pallas_call(
        body,
        grid=(n // _BR,),
        in_specs=[
            pl.BlockSpec((NC, _BR, d), lambda i: (0, i, 0)),
            pl.BlockSpec((_BR, d), lambda i: (i, 0)),
            pl.BlockSpec((_BR, 1), lambda i: (i, 0)),
            pl.BlockSpec((1, d), lambda i: (0, 0)),
            pl.BlockSpec((d, dout), lambda i: (0, 0)),
            pl.BlockSpec((_BR, x.shape[1]), lambda i: (i, 0)),
            pl.BlockSpec((x.shape[1], dout), lambda i: (0, 0)),
            pl.BlockSpec((1, dout), lambda i: (0, 0)),
            pl.BlockSpec((1, dout), lambda i: (0, 0)),
        ],
        out_specs=pl.BlockSpec((_BR, dout), lambda i: (i, 0)),
        out_shape=jax.ShapeDtypeStruct((n, dout), jnp.float32),
    )(parts, g2, dinv, b2, Wf, x, Ws, bf, bs)


def kernel(x, edge_index, W1, b1, W2, b2, Wf, bf, Ws, bs):
    n, d = x.shape
    e = edge_index.shape[1]

    tile_e = 4 * CH
    per_tile = -(-e // (NW * tile_e)) * tile_e
    e_pad = per_tile * NW
    n_rows = per_tile // CH
    ei = edge_index.astype(jnp.int32)
    src3 = jnp.pad(ei[0], (0, e_pad - e)).reshape(NW, n_rows, CH)
    dst3 = jnp.pad(ei[1], (0, e_pad - e),
                   constant_values=n).reshape(NW, n_rows, CH)

    n_pad_deg = -(-(n + 1) // (128 * NS)) * (128 * NS)
    n_pad_agg = -(-(n + 1) // (8 * NS)) * (8 * NS)
    zeros2d = jnp.zeros((n_pad_agg // NS, d), jnp.float32)
    zeros1d = jnp.zeros((n_pad_deg // NS,), jnp.float32)
    ones1d = jnp.ones((CH,), jnp.float32)

    degp = _degree(dst3, zeros1d, ones1d, n_pad_deg)
    degp3 = degp.reshape(NC, n_pad_deg, 1)
    g1, dinv = _tc_prep(x, W1, degp3)
    parts1 = _edge_aggregate(g1, src3, dst3, zeros2d, n_pad_agg)
    g2 = _tc_mid(parts1, g1, dinv, b1.reshape(1, -1), W2)
    parts2 = _edge_aggregate(g2, src3, dst3, zeros2d, n_pad_agg)
    return _tc_final(parts2, g2, dinv, b2.reshape(1, -1), Wf, x, Ws,
                     bf.reshape(1, -1), bs.reshape(1, -1))


# speedup vs baseline: 2.3165x; 2.3165x over previous
import functools

import jax
import jax.numpy as jnp
from jax import lax
from jax.experimental import pallas as pl
from jax.experimental.pallas import tpu as pltpu
from jax.experimental.pallas import tpu_sc as plsc

NC, NS, LANES = 2, 16, 16
NW = NC * NS
CH = 128
HD = 64


def _edge_aggregate(gsplit, srcs, dsts, zeros2d, n_pad):
    n_rows = srcs.shape[1]
    rows_per_tile = n_pad // NS
    half = n_rows // 2
    assert n_rows % 4 == 0

    mesh = plsc.VectorSubcoreMesh(core_axis_name="c", subcore_axis_name="s")

    @functools.partial(
        pl.kernel,
        out_type=jax.ShapeDtypeStruct((NC, n_pad, HD), jnp.float32),
        mesh=mesh,
        scratch_types=[
            pltpu.VMEM((n_rows, CH), jnp.int32),
            pltpu.VMEM((half, CH), jnp.int32),
            pltpu.VMEM((CH, HD), jnp.float32),
            pltpu.VMEM((CH, HD), jnp.float32),
            pltpu.VMEM_SHARED((n_pad, HD), jnp.float32),
            pltpu.VMEM_SHARED((n_pad, HD), jnp.float32),
            pltpu.SemaphoreType.DMA,
            pltpu.SemaphoreType.DMA,
        ],
        compiler_params=pltpu.CompilerParams(use_tc_tiling_on_sc=False),
    )
    def body(g_hbm, src_hbm, dst_hbm, z_hbm, out_hbm, srcv, dstv,
             bufa, bufb, gsp, acc, sema, semb):
        c = lax.axis_index("c")
        s = lax.axis_index("s")
        pltpu.sync_copy(src_hbm.at[s], srcv)
        pltpu.sync_copy(dst_hbm.at[s, pl.ds(0, half)], dstv)
        base = s * rows_per_tile
        pltpu.sync_copy(z_hbm, acc.at[pl.ds(base, rows_per_tile)])
        pltpu.sync_copy(g_hbm.at[c, pl.ds(base, rows_per_tile)],
                        gsp.at[pl.ds(base, rows_per_tile)])
        plsc.subcore_barrier()

        def fire(j, buf, sem):
            pltpu.async_copy(gsp.at[srcv.at[j]], buf, sem)

        def drain(j, buf, sem):
            pltpu.make_async_copy(gsp.at[srcv.at[j]], buf, sem).wait()

        def half_loop(j_base):
            def step(j2, carry):
                j = j_base + 2 * j2
                fire(j + 1, bufb, semb)
                drain(j, bufa, sema)
                pltpu.sync_copy(bufa, acc.at[dstv.at[2 * j2]], add=True)

                @pl.when(j + 2 < n_rows)
                def _():
                    fire(j + 2, bufa, sema)

                drain(j + 1, bufb, semb)
                pltpu.sync_copy(bufb, acc.at[dstv.at[2 * j2 + 1]], add=True)
                return carry

            lax.fori_loop(0, half // 2, step, 0)

        fire(0, bufa, sema)
        half_loop(0)
        pltpu.sync_copy(dst_hbm.at[s, pl.ds(half, half)], dstv)
        half_loop(half)

        plsc.subcore_barrier()
        pltpu.sync_copy(acc.at[pl.ds(base, rows_per_tile)],
                        out_hbm.at[c, pl.ds(base, rows_per_tile)])

    return body(gsplit, srcs, dsts, zeros2d)


def _degree(dst3, zeros1d, ones1d, n_pad):
    n_rows = dst3.shape[1]
    rows_per_tile = n_pad // NS

    mesh = plsc.VectorSubcoreMesh(core_axis_name="c", subcore_axis_name="s")

    @functools.partial(
        pl.kernel,
        out_type=jax.ShapeDtypeStruct((NC, n_pad), jnp.float32),
        mesh=mesh,
        scratch_types=[
            pltpu.VMEM((n_rows, CH), jnp.int32),
            pltpu.VMEM((CH,), jnp.float32),
            pltpu.VMEM_SHARED((n_pad,), jnp.float32),
        ],
    )
    def body(dst_hbm, z_hbm, ones_hbm, out_hbm, dstv, onesv, acc):
        c = lax.axis_index("c")
        s = lax.axis_index("s")
        wid = c * NS + s
        pltpu.sync_copy(ones_hbm, onesv)
        pltpu.sync_copy(dst_hbm.at[wid], dstv)
        base = s * rows_per_tile
        pltpu.sync_copy(z_hbm, acc.at[pl.ds(base, rows_per_tile)])
        plsc.subcore_barrier()

        def chunk(j, carry):
            pltpu.sync_copy(onesv, acc.at[dstv.at[j]], add=True)
            return carry

        lax.fori_loop(0, n_rows, chunk, 0)
        plsc.subcore_barrier()
        pltpu.sync_copy(acc.at[pl.ds(base, rows_per_tile)],
                        out_hbm.at[c, pl.ds(base, rows_per_tile)])

    return body(dst3, zeros1d, ones1d)


_BR = 1000


def _tc_prep(x, W1, degp, n_pad):
    n, d = x.shape

    def body(x_ref, w_ref, degp_ref, g_ref, dinv_ref):
        deg = degp_ref[0] + degp_ref[1] + 1.0
        dinv = 1.0 / jnp.sqrt(deg)
        dinv_ref[...] = dinv
        g = jnp.dot(x_ref[...], w_ref[...],
                    preferred_element_type=jnp.float32) * dinv
        g_ref[0] = g[:, :HD]
        g_ref[1] = g[:, HD:]

    return pl.pallas_call(
        body,
        grid=(n // _BR,),
        in_specs=[
            pl.BlockSpec((_BR, d), lambda i: (i, 0)),
            pl.BlockSpec((d, d), lambda i: (0, 0)),
            pl.BlockSpec((NC, _BR, 1), lambda i: (0, i, 0)),
        ],
        out_specs=[
            pl.BlockSpec((NC, _BR, HD), lambda i: (0, i, 0)),
            pl.BlockSpec((_BR, 1), lambda i: (i, 0)),
        ],
        out_shape=[
            jax.ShapeDtypeStruct((NC, n_pad, HD), jnp.float32),
            jax.ShapeDtypeStruct((n, 1), jnp.float32),
        ],
    )(x, W1, degp)


def _tc_mid(parts, g1, dinv, b1, W2, n_pad):
    n = dinv.shape[0]
    d = W2.shape[0]

    def body(p_ref, g_ref, dinv_ref, b_ref, w_ref, out_ref):
        sm = jnp.concatenate([p_ref[0] + g_ref[0], p_ref[1] + g_ref[1]],
                             axis=1)
        x2 = jnp.maximum(sm * dinv_ref[...] + b_ref[...], 0.0)
        g2 = jnp.dot(x2, w_ref[...],
                     preferred_element_type=jnp.float32) * dinv_ref[...]
        out_ref[0] = g2[:, :HD]
        out_ref[1] = g2[:, HD:]

    return pl.pallas_call(
        body,
        grid=(n // _BR,),
        in_specs=[
            pl.BlockSpec((NC, _BR, HD), lambda i: (0, i, 0)),
            pl.BlockSpec((NC, _BR, HD), lambda i: (0, i, 0)),
            pl.BlockSpec((_BR, 1), lambda i: (i, 0)),
            pl.BlockSpec((1, d), lambda i: (0, 0)),
            pl.BlockSpec((d, d), lambda i: (0, 0)),
        ],
        out_specs=pl.BlockSpec((NC, _BR, HD), lambda i: (0, i, 0)),
        out_shape=jax.ShapeDtypeStruct((NC, n_pad, HD), jnp.float32),
    )(parts, g1, dinv, b1, W2)


def _tc_final(parts, g2, dinv, b2, Wf, x, Ws, bf, bs):
    n = dinv.shape[0]
    d = Wf.shape[0]
    dout = Wf.shape[1]

    def body(p_ref, g_ref, dinv_ref, b2_ref, wf_ref, x_ref, ws_ref,
             bf_ref, bs_ref, out_ref):
        sm = jnp.concatenate([p_ref[0] + g_ref[0], p_ref[1] + g_ref[1]],
                             axis=1)
        h2 = jnp.maximum(sm * dinv_ref[...] + b2_ref[...], 0.0)
        out_ref[...] = (jnp.dot(h2, wf_ref[...], preferred_element_type=jnp.float32)
                        + jnp.dot(x_ref[...], ws_ref[...],
                                  preferred_element_type=jnp.float32)
                        + bf_ref[...] + bs_ref[...])

    return pl.pallas_call(
        body,
        grid=(n // _BR,),
        in_specs=[
            pl.BlockSpec((NC, _BR, HD), lambda i: (0, i, 0)),
            pl.BlockSpec((NC, _BR, HD), lambda i: (0, i, 0)),
            pl.BlockSpec((_BR, 1), lambda i: (i, 0)),
            pl.BlockSpec((1, d), lambda i: (0, 0)),
            pl.BlockSpec((d, dout), lambda i: (0, 0)),
            pl.BlockSpec((_BR, x.shape[1]), lambda i: (i, 0)),
            pl.BlockSpec((x.shape[1], dout), lambda i: (0, 0)),
            pl.BlockSpec((1, dout), lambda i: (0, 0)),
            pl.BlockSpec((1, dout), lambda i: (0, 0)),
        ],
        out_specs=pl.BlockSpec((_BR, dout), lambda i: (i, 0)),
        out_shape=jax.ShapeDtypeStruct((n, dout), jnp.float32),
    )(parts, g2, dinv, b2, Wf, x, Ws, bf, bs)


def kernel(x, edge_index, W1, b1, W2, b2, Wf, bf, Ws, bs):
    n, d = x.shape
    e = edge_index.shape[1]

    tile_e = 4 * CH
    per_tile = -(-e // (NS * tile_e)) * tile_e
    e_pad = per_tile * NS
    n_rows = per_tile // CH
    ei = edge_index.astype(jnp.int32)
    src_pad = jnp.pad(ei[0], (0, e_pad - e))
    dst_pad = jnp.pad(ei[1], (0, e_pad - e), constant_values=n)
    srcs = src_pad.reshape(NS, n_rows, CH)
    dsts = dst_pad.reshape(NS, n_rows, CH)
    dst3 = dst_pad.reshape(NW, e_pad // (NW * CH), CH)

    n_pad_deg = -(-(n + 1) // (128 * NS)) * (128 * NS)
    n_pad_agg = -(-(n + 1) // (8 * NS)) * (8 * NS)
    zeros2d = jnp.zeros((n_pad_agg // NS, HD), jnp.float32)
    zeros1d = jnp.zeros((n_pad_deg // NS,), jnp.float32)
    ones1d = jnp.ones((CH,), jnp.float32)

    degp = _degree(dst3, zeros1d, ones1d, n_pad_deg)
    degp3 = degp.reshape(NC, n_pad_deg, 1)
    g1, dinv = _tc_prep(x, W1, degp3, n_pad_agg)
    parts1 = _edge_aggregate(g1, srcs, dsts, zeros2d, n_pad_agg)
    g2 = _tc_mid(parts1, g1, dinv, b1.reshape(1, -1), W2, n_pad_agg)
    parts2 = _edge_aggregate(g2, srcs, dsts, zeros2d, n_pad_agg)
    return _tc_final(parts2, g2, dinv, b2.reshape(1, -1), Wf, x, Ws,
                     bf.reshape(1, -1), bs.reshape(1, -1))
